# speedup vs baseline: 3.2329x; 3.2329x over previous
import functools

import jax
import jax.numpy as jnp
from jax import lax
from jax.experimental import pallas as pl
from jax.experimental.pallas import tpu as pltpu
from jax.experimental.pallas import tpu_sc as plsc

D_FEAT = 128
D_EDGE = 16
OUT_DIM = 128
_SC_BLOCK = 128


def _nodeproj_body(nf, ws, wd, ps, pd):
    x = nf[...]
    ps[...] = jnp.dot(x, ws[...], preferred_element_type=jnp.float32)
    pd[...] = jnp.dot(x, wd[...], preferred_element_type=jnp.float32)


def _node_projections(node_feat, W_s, W_d):
    N = node_feat.shape[0]
    BLK = 2000
    return pl.pallas_call(
        _nodeproj_body,
        grid=(N // BLK,),
        in_specs=[
            pl.BlockSpec((BLK, D_FEAT), lambda i: (i, 0)),
            pl.BlockSpec((D_FEAT, OUT_DIM), lambda i: (0, 0)),
            pl.BlockSpec((D_FEAT, OUT_DIM), lambda i: (0, 0)),
        ],
        out_specs=[
            pl.BlockSpec((BLK, OUT_DIM), lambda i: (i, 0)),
            pl.BlockSpec((BLK, OUT_DIM), lambda i: (i, 0)),
        ],
        out_shape=[
            jax.ShapeDtypeStruct((N, OUT_DIM), jnp.float32),
            jax.ShapeDtypeStruct((N, OUT_DIM), jnp.float32),
        ],
    )(node_feat, W_s, W_d)


def _make_gather_sum(E):
    info = plsc.get_sparse_core_info()
    NC, NS = info.num_cores, info.num_subcores
    NW = NC * NS
    B = _SC_BLOCK
    nblk = E // B
    mesh = plsc.VectorSubcoreMesh(core_axis_name="c", subcore_axis_name="s")

    @functools.partial(
        pl.kernel,
        mesh=mesh,
        out_type=jax.ShapeDtypeStruct((E, OUT_DIM), jnp.float32),
        scratch_types=[
            pltpu.VMEM((B,), jnp.int32),
            pltpu.VMEM((B,), jnp.int32),
            pltpu.VMEM((B, OUT_DIM), jnp.float32),
            pltpu.VMEM((B, OUT_DIM), jnp.float32),
            pltpu.SemaphoreType.DMA,
            pltpu.SemaphoreType.DMA,
        ],
    )
    def gather_sum(ps_hbm, pd_hbm, src_hbm, dst_hbm, g_hbm,
                   idx_s, idx_d, buf_s, buf_d, sem_s, sem_d):
        wid = lax.axis_index("s") * NC + lax.axis_index("c")
        my_n = (nblk - wid + NW - 1) // NW

        def blk_body(i, carry):
            base = (wid + i * NW) * B
            pltpu.sync_copy(src_hbm.at[pl.ds(base, B)], idx_s)
            pltpu.sync_copy(dst_hbm.at[pl.ds(base, B)], idx_d)
            c1 = pltpu.async_copy(ps_hbm.at[idx_s], buf_s, sem_s)
            c2 = pltpu.async_copy(pd_hbm.at[idx_d], buf_d, sem_d)
            c1.wait()
            c2.wait()

            def row_body(r, rcarry):
                for c in range(OUT_DIM // 16):
                    sl = pl.ds(c * 16, 16)
                    plsc.addupdate(buf_s.at[r, sl], buf_d[r, sl])
                return rcarry

            lax.fori_loop(0, B, row_body, 0)
            pltpu.sync_copy(buf_s, g_hbm.at[pl.ds(base, B)])
            return carry

        lax.fori_loop(0, my_n, blk_body, 0)

    return gather_sum


def _edge_body(ea, we, bb, g, out):
    out[...] = g[...] + jnp.dot(ea[...], we[...],
                                preferred_element_type=jnp.float32) + bb[...]


def _edge_combine(edge_attr, W_e, b2d, G):
    E = edge_attr.shape[0]
    BLK = 4000
    return pl.pallas_call(
        _edge_body,
        grid=(E // BLK,),
        in_specs=[
            pl.BlockSpec((BLK, D_EDGE), lambda i: (i, 0)),
            pl.BlockSpec((D_EDGE, OUT_DIM), lambda i: (0, 0)),
            pl.BlockSpec((1, OUT_DIM), lambda i: (0, 0)),
            pl.BlockSpec((BLK, OUT_DIM), lambda i: (i, 0)),
        ],
        out_specs=pl.BlockSpec((BLK, OUT_DIM), lambda i: (i, 0)),
        out_shape=jax.ShapeDtypeStruct((E, OUT_DIM), jnp.float32),
    )(edge_attr, W_e, b2d, G)


def kernel(node_feat, edge_attr, edge_index, W, b):
    W_e = W[:D_EDGE]
    W_s = W[D_EDGE:D_EDGE + D_FEAT]
    W_d = W[D_EDGE + D_FEAT:]
    src = edge_index[0]
    dst = edge_index[1]
    ps, pd = _node_projections(node_feat, W_s, W_d)
    G = _make_gather_sum(edge_attr.shape[0])(ps, pd, src, dst)
    return _edge_combine(edge_attr, W_e, b.reshape(1, OUT_DIM), G)
